# baseline (speedup 1.0000x reference)
import jax
import jax.numpy as jnp
from jax import lax
from jax.experimental import pallas as pl
from jax.experimental.pallas import tpu as pltpu
from jax.experimental.pallas import tpu_sc as plsc

B = 50000
D = 128
S = 10
NCLS = 40
NC, NS = 2, 16
NW = NC * NS
B_PAD = 51200
C = 32
H0, H1 = 40960, 10240
CORE0_FRAC = 0.6
PAD_ALLOC = B_PAD + 512
NG = 4
GI = C * S // NG


def _sage_gather_body(band, bpw0, nodes_h, adjf_h, feat_h, self_h, nsum_h,
                      ids_v, nidx0, nidx1, nbid0, nbid1,
                      rows0, rows1, sum0, sum1,
                      sem_a, sem_g0, sem_g1, sem_w0, sem_w1):
    nch0 = bpw0 // C
    nch1 = (band - bpw0) // C
    nidx = (nidx0, nidx1)
    nbid = (nbid0, nbid1)
    rows = (rows0, rows1)
    sums = (sum0, sum1)
    sem_g = (sem_g0, sem_g1)
    sem_w = (sem_w0, sem_w1)

    cid = lax.axis_index("c")
    base = lax.axis_index("s") * band + cid * bpw0
    nch = nch0 + cid * (nch1 - nch0)
    pltpu.sync_copy(nodes_h.at[pl.ds(base, max(bpw0, band - bpw0))], ids_v)

    def expand(j, p):
        for i in range(C // 16):
            idvec = ids_v[pl.ds(j * C + i * 16, 16)]
            for s in range(S):
                nidx[p][pl.ds(s * C + i * 16, 16)] = idvec * S + s

    def adj_cps(p):
        return [pltpu.make_async_copy(
                    adjf_h.at[nidx[p].at[pl.ds(g * GI, GI)]],
                    nbid[p].at[pl.ds(g * GI, GI)], sem_a)
                for g in range(NG)]

    def feat_cps(j, p):
        cps = [pltpu.make_async_copy(
                   feat_h.at[nbid[p].at[pl.ds(g * GI, GI)]],
                   rows[p].at[pl.ds(g * GI, GI)], sem_g[p])
               for g in range(NG)]
        cps.append(pltpu.make_async_copy(
            feat_h.at[ids_v.at[pl.ds(j * C, C)]],
            rows[p].at[pl.ds(C * S, C)], sem_g[p]))
        return cps

    def write_cps(j, p):
        row0 = base + j * C
        return [pltpu.make_async_copy(rows[p].at[pl.ds(C * S, C)],
                                      self_h.at[pl.ds(row0, C)], sem_w[p]),
                pltpu.make_async_copy(sums[p], nsum_h.at[pl.ds(row0, C)],
                                      sem_w[p])]

    expand(0, 0)
    for cp in adj_cps(0):
        cp.start()
    for cp in adj_cps(0):
        cp.wait()
    for cp in feat_cps(0, 0):
        cp.start()
    expand(1, 1)
    for cp in adj_cps(1):
        cp.start()

    def outer_body(kk, carry):
        for b in range(2):
            k = kk * 2 + b
            pb = b
            pn = 1 - b

            def s1():
                @pl.when(k >= 1)
                def _():
                    for cp in write_cps(k - 1, pn):
                        cp.wait()
                for cp in adj_cps(pn):
                    cp.wait()
                for cp in feat_cps(k + 1, pn):
                    cp.start()

            @pl.when(k + 1 < nch)
            def _():
                s1()

            @pl.when(k + 2 < nch)
            def _():
                expand(k + 2, pb)
                for cp in adj_cps(pb):
                    cp.start()

            for cp in feat_cps(k, pb):
                cp.wait()

            def red_body(c, carry2):
                for d in range(D // 16):
                    sl = pl.ds(d * 16, 16)
                    acc = rows[pb][c, sl]
                    for s in range(1, S):
                        acc = acc + rows[pb][s * C + c, sl]
                    sums[pb][c, sl] = acc
                return carry2

            lax.fori_loop(0, C, red_body, 0)
            for cp in write_cps(k, pb):
                cp.start()
        return carry

    lax.fori_loop(0, nch // 2, outer_body, 0)
    for cp in write_cps(nch - 2, 0):
        cp.wait()
    for cp in write_cps(nch - 1, 1):
        cp.wait()


def _make_gather(h):
    import functools
    band = h // NS
    bpw0 = int(band * CORE0_FRAC) // (2 * C) * (2 * C)
    body = functools.partial(_sage_gather_body, band, bpw0)
    return pl.kernel(
        body,
        out_type=(jax.ShapeDtypeStruct((h, D), jnp.float32),
                  jax.ShapeDtypeStruct((h, D), jnp.float32)),
        mesh=plsc.VectorSubcoreMesh(core_axis_name="c", subcore_axis_name="s"),
        scratch_types=[
            pltpu.VMEM((max(bpw0, band - bpw0),), jnp.int32),
            pltpu.VMEM((C * S,), jnp.int32),
            pltpu.VMEM((C * S,), jnp.int32),
            pltpu.VMEM((C * S,), jnp.int32),
            pltpu.VMEM((C * S,), jnp.int32),
            pltpu.VMEM((C * S + C, D), jnp.float32),
            pltpu.VMEM((C * S + C, D), jnp.float32),
            pltpu.VMEM((C, D), jnp.float32),
            pltpu.VMEM((C, D), jnp.float32),
            pltpu.SemaphoreType.DMA,
            pltpu.SemaphoreType.DMA,
            pltpu.SemaphoreType.DMA,
            pltpu.SemaphoreType.DMA,
            pltpu.SemaphoreType.DMA,
        ],
    )


def _mlp_body(inv_ref, self_ref, sum_ref, w1_ref, w2_ref, wc_ref, out_ref):
    inv = inv_ref[0]
    w1 = w1_ref[...].astype(jnp.bfloat16)
    w2 = (w2_ref[...] * inv).astype(jnp.bfloat16)
    h = jnp.dot(self_ref[...].astype(jnp.bfloat16), w1,
                preferred_element_type=jnp.float32)
    h = h + jnp.dot(sum_ref[...].astype(jnp.bfloat16), w2,
                    preferred_element_type=jnp.float32)
    h = jnp.maximum(h, 0.0)
    out_ref[...] = jnp.dot(h, wc_ref[...], preferred_element_type=jnp.float32)


BM = 3200


def _mlp_call(inv, self_f, nsum, W_sage, W_cls):
    h = self_f.shape[0]
    grid = (h // BM,)
    return pl.pallas_call(
        _mlp_body,
        grid=grid,
        in_specs=[
            pl.BlockSpec(memory_space=pltpu.SMEM),
            pl.BlockSpec((BM, D), lambda i: (i, 0)),
            pl.BlockSpec((BM, D), lambda i: (i, 0)),
            pl.BlockSpec((D, D), lambda i: (0, 0)),
            pl.BlockSpec((D, D), lambda i: (0, 0)),
            pl.BlockSpec((D, NCLS), lambda i: (0, 0)),
        ],
        out_specs=pl.BlockSpec((BM, NCLS), lambda i: (i, 0)),
        out_shape=jax.ShapeDtypeStruct((h, NCLS), jnp.float32),
    )(inv, self_f, nsum, W_sage[:D], W_sage[D:], W_cls)


def kernel(nodes, adj, sample, feat, W_sage, W_cls):
    nodes_p = jnp.pad(nodes, (0, PAD_ALLOC - B))
    adj_flat = adj.reshape(-1)
    inv = jnp.reshape(1.0 / jnp.asarray(sample, jnp.float32), (1,))
    self0, nsum0 = _make_gather(H0)(nodes_p[:H0 + 512], adj_flat, feat)
    self1, nsum1 = _make_gather(H1)(nodes_p[H0:], adj_flat, feat)
    out0 = _mlp_call(inv, self0, nsum0, W_sage, W_cls)
    out1 = _mlp_call(inv, self1, nsum1, W_sage, W_cls)
    return jnp.concatenate([out0, out1], axis=0)[:B]


# speedup vs baseline: 1.0342x; 1.0342x over previous
import jax
import jax.numpy as jnp
from jax import lax
from jax.experimental import pallas as pl
from jax.experimental.pallas import tpu as pltpu
from jax.experimental.pallas import tpu_sc as plsc

B = 50000
D = 128
S = 10
NCLS = 40
NC, NS = 2, 16
NW = NC * NS
B_PAD = 51200
H = B_PAD // 2
C = 32
BAND = H // NS
BPW0 = 960
BPW1 = BAND - BPW0
NCH0 = BPW0 // C
NCH1 = BPW1 // C
SLACK = abs(BPW0 - BPW1)
PAD_ALLOC = B_PAD + SLACK
NG = 4
GI = C * S // NG


def _sage_gather_body(nodes_h, adjf_h, feat_h, self_h, nsum_h,
                      ids_v, nidx0, nidx1, nbid0, nbid1,
                      rows0, rows1, sum0, sum1,
                      sem_a, sem_g0, sem_g1, sem_w0, sem_w1):
    nidx = (nidx0, nidx1)
    nbid = (nbid0, nbid1)
    rows = (rows0, rows1)
    sums = (sum0, sum1)
    sem_g = (sem_g0, sem_g1)
    sem_w = (sem_w0, sem_w1)

    cid = lax.axis_index("c")
    base = lax.axis_index("s") * BAND + cid * BPW0
    nch = NCH0 + cid * (NCH1 - NCH0)
    pltpu.sync_copy(nodes_h.at[pl.ds(base, max(BPW0, BPW1))], ids_v)

    def expand(j, p):
        for i in range(C // 16):
            idvec = ids_v[pl.ds(j * C + i * 16, 16)]
            for s in range(S):
                nidx[p][pl.ds(s * C + i * 16, 16)] = idvec * S + s

    def adj_cps(p):
        return [pltpu.make_async_copy(
                    adjf_h.at[nidx[p].at[pl.ds(g * GI, GI)]],
                    nbid[p].at[pl.ds(g * GI, GI)], sem_a)
                for g in range(NG)]

    def feat_cps(j, p):
        cps = [pltpu.make_async_copy(
                   feat_h.at[nbid[p].at[pl.ds(g * GI, GI)]],
                   rows[p].at[pl.ds(g * GI, GI)], sem_g[p])
               for g in range(NG)]
        cps.append(pltpu.make_async_copy(
            feat_h.at[ids_v.at[pl.ds(j * C, C)]],
            rows[p].at[pl.ds(C * S, C)], sem_g[p]))
        return cps

    def write_cps(j, p):
        row0 = base + j * C
        return [pltpu.make_async_copy(rows[p].at[pl.ds(C * S, C)],
                                      self_h.at[pl.ds(row0, C)], sem_w[p]),
                pltpu.make_async_copy(sums[p], nsum_h.at[pl.ds(row0, C)],
                                      sem_w[p])]

    expand(0, 0)
    for cp in adj_cps(0):
        cp.start()
    for cp in adj_cps(0):
        cp.wait()
    for cp in feat_cps(0, 0):
        cp.start()
    expand(1, 1)
    for cp in adj_cps(1):
        cp.start()

    def outer_body(kk, carry):
        for b in range(2):
            k = kk * 2 + b
            pb = b
            pn = 1 - b

            def s1():
                @pl.when(k >= 1)
                def _():
                    for cp in write_cps(k - 1, pn):
                        cp.wait()
                for cp in adj_cps(pn):
                    cp.wait()
                for cp in feat_cps(k + 1, pn):
                    cp.start()

            @pl.when(k + 1 < nch)
            def _():
                s1()

            @pl.when(k + 2 < nch)
            def _():
                expand(k + 2, pb)
                for cp in adj_cps(pb):
                    cp.start()

            for cp in feat_cps(k, pb):
                cp.wait()

            def red_body(c, carry2):
                for d in range(D // 16):
                    sl = pl.ds(d * 16, 16)
                    acc = rows[pb][c, sl]
                    for s in range(1, S):
                        acc = acc + rows[pb][s * C + c, sl]
                    sums[pb][c, sl] = acc
                return carry2

            lax.fori_loop(0, C, red_body, 0)
            for cp in write_cps(k, pb):
                cp.start()
        return carry

    lax.fori_loop(0, nch // 2, outer_body, 0)
    for cp in write_cps(nch - 2, 0):
        cp.wait()
    for cp in write_cps(nch - 1, 1):
        cp.wait()


def _make_gather():
    return pl.kernel(
        _sage_gather_body,
        out_type=(jax.ShapeDtypeStruct((H, D), jnp.float32),
                  jax.ShapeDtypeStruct((H, D), jnp.float32)),
        mesh=plsc.VectorSubcoreMesh(core_axis_name="c", subcore_axis_name="s"),
        scratch_types=[
            pltpu.VMEM((max(BPW0, BPW1),), jnp.int32),
            pltpu.VMEM((C * S,), jnp.int32),
            pltpu.VMEM((C * S,), jnp.int32),
            pltpu.VMEM((C * S,), jnp.int32),
            pltpu.VMEM((C * S,), jnp.int32),
            pltpu.VMEM((C * S + C, D), jnp.float32),
            pltpu.VMEM((C * S + C, D), jnp.float32),
            pltpu.VMEM((C, D), jnp.float32),
            pltpu.VMEM((C, D), jnp.float32),
            pltpu.SemaphoreType.DMA,
            pltpu.SemaphoreType.DMA,
            pltpu.SemaphoreType.DMA,
            pltpu.SemaphoreType.DMA,
            pltpu.SemaphoreType.DMA,
        ],
    )


def _mlp_body(inv_ref, self_ref, sum_ref, w1_ref, w2_ref, wc_ref, out_ref):
    inv = inv_ref[0]
    w1 = w1_ref[...].astype(jnp.bfloat16)
    w2 = (w2_ref[...] * inv).astype(jnp.bfloat16)
    h = jnp.dot(self_ref[...].astype(jnp.bfloat16), w1,
                preferred_element_type=jnp.float32)
    h = h + jnp.dot(sum_ref[...].astype(jnp.bfloat16), w2,
                    preferred_element_type=jnp.float32)
    h = jnp.maximum(h, 0.0)
    out_ref[...] = jnp.dot(h, wc_ref[...], preferred_element_type=jnp.float32)


BM = 3200


def _mlp_call(inv, self_f, nsum, W_sage, W_cls):
    grid = (H // BM,)
    return pl.pallas_call(
        _mlp_body,
        grid=grid,
        in_specs=[
            pl.BlockSpec(memory_space=pltpu.SMEM),
            pl.BlockSpec((BM, D), lambda i: (i, 0)),
            pl.BlockSpec((BM, D), lambda i: (i, 0)),
            pl.BlockSpec((D, D), lambda i: (0, 0)),
            pl.BlockSpec((D, D), lambda i: (0, 0)),
            pl.BlockSpec((D, NCLS), lambda i: (0, 0)),
        ],
        out_specs=pl.BlockSpec((BM, NCLS), lambda i: (i, 0)),
        out_shape=jax.ShapeDtypeStruct((H, NCLS), jnp.float32),
    )(inv, self_f, nsum, W_sage[:D], W_sage[D:], W_cls)


def kernel(nodes, adj, sample, feat, W_sage, W_cls):
    nodes_p = jnp.pad(nodes, (0, PAD_ALLOC - B))
    adj_flat = adj.reshape(-1)
    inv = jnp.reshape(1.0 / jnp.asarray(sample, jnp.float32), (1,))
    gather = _make_gather()
    self0, nsum0 = gather(nodes_p[:H + SLACK], adj_flat, feat)
    self1, nsum1 = gather(nodes_p[H:], adj_flat, feat)
    out0 = _mlp_call(inv, self0, nsum0, W_sage, W_cls)
    out1 = _mlp_call(inv, self1, nsum1, W_sage, W_cls)
    return jnp.concatenate([out0, out1], axis=0)[:B]


# speedup vs baseline: 1.0410x; 1.0066x over previous
import jax
import jax.numpy as jnp
from jax import lax
from jax.experimental import pallas as pl
from jax.experimental.pallas import tpu as pltpu
from jax.experimental.pallas import tpu_sc as plsc

B = 50000
D = 128
S = 10
NCLS = 40
NC, NS = 2, 16
NW = NC * NS
B_PAD = 51200
H = B_PAD // 2
C = 32
BAND = H // NS
BPW0 = 960
BPW1 = BAND - BPW0
NCH0 = BPW0 // C
NCH1 = BPW1 // C
SLACK = abs(BPW0 - BPW1)
PAD_ALLOC = B_PAD + SLACK
NG = 4
GI = C * S // NG


def _sage_gather_body(nodes_h, adjf_h, feat_h, self_h, nsum_h,
                      ids_v, nidx0, nidx1, nbid0, nbid1,
                      rows0, rows1, sum0, sum1,
                      sem_a, sem_g0, sem_g1, sem_w0, sem_w1):
    nidx = (nidx0, nidx1)
    nbid = (nbid0, nbid1)
    rows = (rows0, rows1)
    sums = (sum0, sum1)
    sem_g = (sem_g0, sem_g1)
    sem_w = (sem_w0, sem_w1)

    cid = lax.axis_index("c")
    base = lax.axis_index("s") * BAND + cid * BPW0
    nch = NCH0 + cid * (NCH1 - NCH0)
    pltpu.sync_copy(nodes_h.at[pl.ds(base, max(BPW0, BPW1))], ids_v)

    def expand(j, p):
        for i in range(C // 16):
            idvec = ids_v[pl.ds(j * C + i * 16, 16)]
            for s in range(S):
                nidx[p][pl.ds(s * C + i * 16, 16)] = idvec * S + s

    def adj_cps(p):
        return [pltpu.make_async_copy(
                    adjf_h.at[nidx[p].at[pl.ds(g * GI, GI)]],
                    nbid[p].at[pl.ds(g * GI, GI)], sem_a)
                for g in range(NG)]

    def feat_cps(j, p):
        cps = [pltpu.make_async_copy(
                   feat_h.at[nbid[p].at[pl.ds(g * GI, GI)]],
                   rows[p].at[pl.ds(g * GI, GI)], sem_g[p])
               for g in range(NG)]
        cps.append(pltpu.make_async_copy(
            feat_h.at[ids_v.at[pl.ds(j * C, C)]],
            rows[p].at[pl.ds(C * S, C)], sem_g[p]))
        return cps

    def write_cps(j, p):
        row0 = base + j * C
        return [pltpu.make_async_copy(rows[p].at[pl.ds(C * S, C)],
                                      self_h.at[pl.ds(row0, C)], sem_w[p]),
                pltpu.make_async_copy(sums[p], nsum_h.at[pl.ds(row0, C)],
                                      sem_w[p])]

    expand(0, 0)
    for cp in adj_cps(0):
        cp.start()
    for cp in adj_cps(0):
        cp.wait()
    for cp in feat_cps(0, 0):
        cp.start()
    expand(1, 1)
    for cp in adj_cps(1):
        cp.start()

    def outer_body(kk, carry):
        for b in range(2):
            k = kk * 2 + b
            pb = b
            pn = 1 - b

            def s1():
                @pl.when(k >= 1)
                def _():
                    for cp in write_cps(k - 1, pn):
                        cp.wait()
                for cp in adj_cps(pn):
                    cp.wait()
                for cp in feat_cps(k + 1, pn):
                    cp.start()

            @pl.when(k + 1 < nch)
            def _():
                s1()

            @pl.when(k + 2 < nch)
            def _():
                expand(k + 2, pb)
                for cp in adj_cps(pb):
                    cp.start()

            for cp in feat_cps(k, pb):
                cp.wait()

            def red_body(c, carry2):
                for d in range(D // 16):
                    sl = pl.ds(d * 16, 16)
                    acc = rows[pb][c, sl]
                    for s in range(1, S):
                        acc = acc + rows[pb][s * C + c, sl]
                    sums[pb][c, sl] = acc
                return carry2

            lax.fori_loop(0, C, red_body, 0)
            for cp in write_cps(k, pb):
                cp.start()
        return carry

    lax.fori_loop(0, nch // 2, outer_body, 0)
    for cp in write_cps(nch - 2, 0):
        cp.wait()
    for cp in write_cps(nch - 1, 1):
        cp.wait()


def _make_gather():
    return pl.kernel(
        _sage_gather_body,
        out_type=(jax.ShapeDtypeStruct((H, D), jnp.float32),
                  jax.ShapeDtypeStruct((H, D), jnp.float32)),
        mesh=plsc.VectorSubcoreMesh(core_axis_name="c", subcore_axis_name="s"),
        scratch_types=[
            pltpu.VMEM((max(BPW0, BPW1),), jnp.int32),
            pltpu.VMEM((C * S,), jnp.int32),
            pltpu.VMEM((C * S,), jnp.int32),
            pltpu.VMEM((C * S,), jnp.int32),
            pltpu.VMEM((C * S,), jnp.int32),
            pltpu.VMEM((C * S + C, D), jnp.float32),
            pltpu.VMEM((C * S + C, D), jnp.float32),
            pltpu.VMEM((C, D), jnp.float32),
            pltpu.VMEM((C, D), jnp.float32),
            pltpu.SemaphoreType.DMA,
            pltpu.SemaphoreType.DMA,
            pltpu.SemaphoreType.DMA,
            pltpu.SemaphoreType.DMA,
            pltpu.SemaphoreType.DMA,
        ],
    )


def _mlp_body(inv_ref, self_ref, sum_ref, w1_ref, w2_ref, wc_ref, out_ref):
    inv = inv_ref[0]
    w1 = w1_ref[...].astype(jnp.bfloat16)
    w2 = (w2_ref[...] * inv).astype(jnp.bfloat16)
    h = jnp.dot(self_ref[...].astype(jnp.bfloat16), w1,
                preferred_element_type=jnp.float32)
    h = h + jnp.dot(sum_ref[...].astype(jnp.bfloat16), w2,
                    preferred_element_type=jnp.float32)
    h = jnp.maximum(h, 0.0)
    out_ref[...] = jnp.dot(h, wc_ref[...], preferred_element_type=jnp.float32)


BM = 6400


def _mlp_call(inv, self_f, nsum, W_sage, W_cls):
    grid = (H // BM,)
    return pl.pallas_call(
        _mlp_body,
        grid=grid,
        in_specs=[
            pl.BlockSpec(memory_space=pltpu.SMEM),
            pl.BlockSpec((BM, D), lambda i: (i, 0)),
            pl.BlockSpec((BM, D), lambda i: (i, 0)),
            pl.BlockSpec((D, D), lambda i: (0, 0)),
            pl.BlockSpec((D, D), lambda i: (0, 0)),
            pl.BlockSpec((D, NCLS), lambda i: (0, 0)),
        ],
        out_specs=pl.BlockSpec((BM, NCLS), lambda i: (i, 0)),
        out_shape=jax.ShapeDtypeStruct((H, NCLS), jnp.float32),
    )(inv, self_f, nsum, W_sage[:D], W_sage[D:], W_cls)


def kernel(nodes, adj, sample, feat, W_sage, W_cls):
    nodes_p = jnp.pad(nodes, (0, PAD_ALLOC - B))
    adj_flat = adj.reshape(-1)
    inv = jnp.reshape(1.0 / jnp.asarray(sample, jnp.float32), (1,))
    gather = _make_gather()
    self0, nsum0 = gather(nodes_p[:H + SLACK], adj_flat, feat)
    self1, nsum1 = gather(nodes_p[H:], adj_flat, feat)
    out0 = _mlp_call(inv, self0, nsum0, W_sage, W_cls)
    out1 = _mlp_call(inv, self1, nsum1, W_sage, W_cls)
    return jnp.concatenate([out0, out1], axis=0)[:B]
